# Initial kernel scaffold; baseline (speedup 1.0000x reference)
import functools

import jax
import jax.numpy as jnp
from jax import lax
from jax.experimental import pallas as pl
from jax.experimental.pallas import tpu as pltpu
from jax.experimental.pallas import tpu_sc as plsc

N = 50000
EMB = 64
NE = 800000

NC = 2
NS = 16
HALF = N // 2
TBL = 25600
STRIPE = TBL // NS
DUMMY = HALF
DSPREAD = 512
CHUNK = 1024
NSTREAM = CHUNK // 128
NCH = 49
EPC = NS * CHUNK * NCH
BTC = 2000


def _mesh():
    return plsc.VectorSubcoreMesh(
        core_axis_name="c", subcore_axis_name="s", num_cores=NC,
        num_subcores=NS)


def _sc_deg(cols3, ones_hbm, zeros_hbm):
    @functools.partial(
        pl.kernel,
        out_type=jax.ShapeDtypeStruct((NC, TBL, 16), jnp.float32),
        mesh=_mesh(),
        scratch_types=[
            pltpu.VMEM((8, 128), jnp.int32),
            pltpu.VMEM((128, 16), jnp.float32),
            pltpu.VMEM_SHARED((TBL, 16), jnp.float32),
            pltpu.SemaphoreType.DMA,
        ],
    )
    def k(col_hbm, ones_hbm, z_hbm, out_hbm, cidx, ones_v, table, sem):
        cid = lax.axis_index("c")
        sid = lax.axis_index("s")
        pltpu.sync_copy(ones_hbm, ones_v)
        pltpu.sync_copy(z_hbm, table.at[pl.ds(sid * STRIPE, STRIPE)])
        plsc.subcore_barrier()

        @pl.loop(0, NCH)
        def _(ci):
            rb = sid * (NCH * NSTREAM) + ci * NSTREAM
            pltpu.sync_copy(col_hbm.at[cid, pl.ds(rb, NSTREAM)], cidx)
            cps = [
                pltpu.async_copy(ones_v, table.at[cidx.at[j]], sem, add=True)
                for j in range(NSTREAM)
            ]
            for c in cps:
                c.wait()

        plsc.subcore_barrier()
        pltpu.sync_copy(table.at[pl.ds(sid * STRIPE, STRIPE)],
                        out_hbm.at[cid, pl.ds(sid * STRIPE, STRIPE)])

    return k(cols3, ones_hbm, zeros_hbm)


def _sc_prop(xs, rows2, cols3, zeros_hbm):
    @functools.partial(
        pl.kernel,
        out_type=jax.ShapeDtypeStruct((NC, TBL, EMB), jnp.float32),
        mesh=_mesh(),
        scratch_types=[
            pltpu.VMEM((8, 128), jnp.int32),
            pltpu.VMEM((8, 128), jnp.int32),
            pltpu.VMEM((CHUNK, EMB), jnp.float32),
            pltpu.VMEM_SHARED((TBL, EMB), jnp.float32),
            pltpu.SemaphoreType.DMA,
            pltpu.SemaphoreType.DMA,
        ],
    )
    def k(xs_hbm, row_hbm, col_hbm, z_hbm, out_hbm, ridx, cidx, rows_v,
          table, gsem, ssem):
        cid = lax.axis_index("c")
        sid = lax.axis_index("s")
        pltpu.sync_copy(z_hbm, table.at[pl.ds(sid * STRIPE, STRIPE)])
        plsc.subcore_barrier()

        @pl.loop(0, NCH)
        def _(ci):
            rb = sid * (NCH * NSTREAM) + ci * NSTREAM
            pltpu.sync_copy(row_hbm.at[pl.ds(rb, NSTREAM)], ridx)
            pltpu.sync_copy(col_hbm.at[cid, pl.ds(rb, NSTREAM)], cidx)
            gcs = [
                pltpu.async_copy(xs_hbm.at[ridx.at[j]],
                                 rows_v.at[pl.ds(j * 128, 128)], gsem)
                for j in range(NSTREAM)
            ]
            for c in gcs:
                c.wait()
            scs = [
                pltpu.async_copy(rows_v.at[pl.ds(j * 128, 128)],
                                 table.at[cidx.at[j]], ssem, add=True)
                for j in range(NSTREAM)
            ]
            for c in scs:
                c.wait()

        plsc.subcore_barrier()
        pltpu.sync_copy(table.at[pl.ds(sid * STRIPE, STRIPE)],
                        out_hbm.at[cid, pl.ds(sid * STRIPE, STRIPE)])

    return k(xs, rows2, cols3, zeros_hbm)


def _dis_block(deg_blk):
    d = deg_blk[:, 0:1]
    return jnp.where(d > 0, lax.rsqrt(d), 0.0)


def _tc_prescale(E, deg):
    def body(deg_ref, e_ref, o_ref):
        o_ref[...] = e_ref[...] * _dis_block(deg_ref[...])

    return pl.pallas_call(
        body,
        grid=(N // BTC,),
        in_specs=[
            pl.BlockSpec((BTC, 16), lambda i: (i, 0)),
            pl.BlockSpec((BTC, EMB), lambda i: (i, 0)),
        ],
        out_specs=pl.BlockSpec((BTC, EMB), lambda i: (i, 0)),
        out_shape=jax.ShapeDtypeStruct((N, EMB), jnp.float32),
    )(deg, E)


def _tc_post(agg, deg):
    def body(agg_ref, deg_ref, x_ref, xs_ref):
        dis = _dis_block(deg_ref[...])
        t = agg_ref[...] * dis
        t = jnp.where(t >= 0, t, 0.01 * t)
        nrm = jnp.sqrt(jnp.sum(t * t, axis=1, keepdims=True))
        x = t / jnp.maximum(nrm, 1e-12)
        x_ref[...] = x
        xs_ref[...] = x * dis

    return pl.pallas_call(
        body,
        grid=(N // BTC,),
        in_specs=[
            pl.BlockSpec((BTC, EMB), lambda i: (i, 0)),
            pl.BlockSpec((BTC, 16), lambda i: (i, 0)),
        ],
        out_specs=[
            pl.BlockSpec((BTC, EMB), lambda i: (i, 0)),
            pl.BlockSpec((BTC, EMB), lambda i: (i, 0)),
        ],
        out_shape=[
            jax.ShapeDtypeStruct((N, EMB), jnp.float32),
            jax.ShapeDtypeStruct((N, EMB), jnp.float32),
        ],
    )(agg, deg)


def _tc_final(E, x1, x2, x3):
    def body(e_ref, a_ref, b_ref, c_ref, o_ref):
        o_ref[...] = 0.25 * (e_ref[...] + a_ref[...] + b_ref[...]
                             + c_ref[...])

    spec = pl.BlockSpec((BTC, EMB), lambda i: (i, 0))
    return pl.pallas_call(
        body,
        grid=(N // BTC,),
        in_specs=[spec, spec, spec, spec],
        out_specs=spec,
        out_shape=jax.ShapeDtypeStruct((N, EMB), jnp.float32),
    )(E, x1, x2, x3)


def kernel(edge_index, E):
    row = edge_index[0]
    col = edge_index[1]
    pad = EPC - NE

    e = jnp.arange(EPC, dtype=jnp.int32)
    dummy = DUMMY + (e & (DSPREAD - 1))
    colp = jnp.concatenate([col, jnp.full((pad,), -1, jnp.int32)])
    rowp = jnp.concatenate([row, jnp.zeros((pad,), jnp.int32)])
    col0 = jnp.where((colp >= 0) & (colp < HALF), colp, dummy)
    col1 = jnp.where(colp >= HALF, colp - HALF, dummy)
    cols3 = jnp.stack([col0, col1]).reshape(NC, EPC // 128, 128)
    rows2 = rowp.reshape(EPC // 128, 128)

    zeros16 = jnp.zeros((STRIPE, 16), jnp.float32)
    zeros64 = jnp.zeros((STRIPE, EMB), jnp.float32)
    ones16 = jnp.ones((128, 16), jnp.float32)

    degp = _sc_deg(cols3, ones16, zeros16)
    deg = jnp.concatenate([degp[0, :HALF], degp[1, :HALF]], axis=0)

    xs = _tc_prescale(E, deg)
    xlist = []
    for _ in range(3):
        aggp = _sc_prop(xs, rows2, cols3, zeros64)
        agg = jnp.concatenate([aggp[0, :HALF], aggp[1, :HALF]], axis=0)
        x, xs = _tc_post(agg, deg)
        xlist.append(x)

    return _tc_final(E, *xlist)


# speedup vs baseline: 7.4959x; 7.4959x over previous
import functools

import jax
import jax.numpy as jnp
from jax import lax
from jax.experimental import pallas as pl
from jax.experimental.pallas import tpu as pltpu
from jax.experimental.pallas import tpu_sc as plsc

N = 50000
EMB = 64
NE = 800000

NC = 2
NS = 16
HALF = N // 2
TBL = 25088
STRIPE = TBL // NS
DUMMY = HALF
DSPREAD = 64
NBUF = 4
C = 96
M = 524
EPC = NS * C * M
CD = 128
MD = 392
EPD = NS * CD * MD
BTC = 2000


def _mesh():
    return plsc.VectorSubcoreMesh(
        core_axis_name="c", subcore_axis_name="s", num_cores=NC,
        num_subcores=NS)


_SC_PARAMS = pltpu.CompilerParams(use_tc_tiling_on_sc=False,
                                  internal_scratch_in_bytes=131072)


def _sc_deg(colsd, ones16, z16):
    @functools.partial(
        pl.kernel,
        out_type=jax.ShapeDtypeStruct((NC, TBL, 16), jnp.float32),
        mesh=_mesh(),
        scratch_types=[
            pltpu.VMEM((2, CD), jnp.int32),
            pltpu.VMEM((CD, 16), jnp.float32),
            pltpu.VMEM_SHARED((TBL, 16), jnp.float32),
            pltpu.SemaphoreType.DMA,
            pltpu.SemaphoreType.DMA,
        ],
        compiler_params=_SC_PARAMS,
    )
    def k(col_hbm, ones_hbm, z_hbm, out_hbm, cidx, ones_v, table, isem,
          ssem):
        cid = lax.axis_index("c")
        sid = lax.axis_index("s")
        pltpu.sync_copy(ones_hbm, ones_v)
        pltpu.sync_copy(z_hbm, table.at[pl.ds(sid * STRIPE, STRIPE)])
        plsc.subcore_barrier()

        base = sid * MD
        pltpu.sync_copy(col_hbm.at[cid, base], cidx.at[0])

        @pl.loop(0, MD // 2)
        def _(it):
            c1 = pltpu.async_copy(col_hbm.at[cid, base + 2 * it + 1],
                                  cidx.at[1], isem)
            s0 = pltpu.async_copy(ones_v, table.at[cidx.at[0]], ssem,
                                  add=True)
            c1.wait()
            s0.wait()

            @pl.when(it < MD // 2 - 1)
            def _():
                pltpu.async_copy(col_hbm.at[cid, base + 2 * it + 2],
                                 cidx.at[0], isem).wait()

            pltpu.async_copy(ones_v, table.at[cidx.at[1]], ssem,
                             add=True).wait()

        plsc.subcore_barrier()
        pltpu.sync_copy(table.at[pl.ds(sid * STRIPE, STRIPE)],
                        out_hbm.at[cid, pl.ds(sid * STRIPE, STRIPE)])

    return k(colsd, ones16, z16)


def _sc_prop(xs, rows2, cols3, z64):
    @functools.partial(
        pl.kernel,
        out_type=jax.ShapeDtypeStruct((NC, TBL, EMB), jnp.float32),
        mesh=_mesh(),
        scratch_types=[
            pltpu.VMEM((NBUF, C), jnp.int32),
            pltpu.VMEM((NBUF, C), jnp.int32),
            pltpu.VMEM((NBUF, C, EMB), jnp.float32),
            pltpu.VMEM_SHARED((TBL, EMB), jnp.float32),
            pltpu.SemaphoreType.DMA,
            pltpu.SemaphoreType.DMA,
            pltpu.SemaphoreType.DMA,
            pltpu.SemaphoreType.DMA,
            pltpu.SemaphoreType.DMA,
        ],
        compiler_params=_SC_PARAMS,
    )
    def k(xs_hbm, row_hbm, col_hbm, z_hbm, out_hbm, ridx, cidx, rows_v,
          table, g0, g1, g2, g3, ssem):
        cid = lax.axis_index("c")
        sid = lax.axis_index("s")
        gsem = [g0, g1, g2, g3]
        pltpu.sync_copy(z_hbm, table.at[pl.ds(sid * STRIPE, STRIPE)])
        plsc.subcore_barrier()

        base = sid * M
        for b in range(NBUF):
            pltpu.sync_copy(row_hbm.at[base + b], ridx.at[b])
            pltpu.sync_copy(col_hbm.at[cid, base + b], cidx.at[b])
            pltpu.async_copy(xs_hbm.at[ridx.at[b]], rows_v.at[b], gsem[b])

        @pl.loop(0, M // NBUF)
        def _(it):
            ci = it * NBUF
            for b in range(NBUF):
                pltpu.make_async_copy(xs_hbm.at[pl.ds(0, C)], rows_v.at[b],
                                      gsem[b]).wait()
                pltpu.sync_copy(rows_v.at[b], table.at[cidx.at[b]], add=True)

                @pl.when(it < M // NBUF - 1)
                def _():
                    nxt = base + ci + NBUF + b
                    pltpu.sync_copy(row_hbm.at[nxt], ridx.at[b])
                    pltpu.sync_copy(col_hbm.at[cid, nxt], cidx.at[b])
                    pltpu.async_copy(xs_hbm.at[ridx.at[b]], rows_v.at[b],
                                     gsem[b])

        plsc.subcore_barrier()
        pltpu.sync_copy(table.at[pl.ds(sid * STRIPE, STRIPE)],
                        out_hbm.at[cid, pl.ds(sid * STRIPE, STRIPE)])

    return k(xs, rows2, cols3, z64)


def _dis_block(deg_blk):
    d = deg_blk[:, 0:1]
    return jnp.where(d > 0, lax.rsqrt(d), 0.0)


def _tc_prescale(E, deg):
    def body(deg_ref, e_ref, o_ref):
        o_ref[...] = e_ref[...] * _dis_block(deg_ref[...])

    return pl.pallas_call(
        body,
        grid=(N // BTC,),
        in_specs=[
            pl.BlockSpec((BTC, 16), lambda i: (i, 0)),
            pl.BlockSpec((BTC, EMB), lambda i: (i, 0)),
        ],
        out_specs=pl.BlockSpec((BTC, EMB), lambda i: (i, 0)),
        out_shape=jax.ShapeDtypeStruct((N, EMB), jnp.float32),
    )(deg, E)


def _tc_post(agg, deg):
    def body(agg_ref, deg_ref, x_ref, xs_ref):
        dis = _dis_block(deg_ref[...])
        t = agg_ref[...] * dis
        t = jnp.where(t >= 0, t, 0.01 * t)
        nrm = jnp.sqrt(jnp.sum(t * t, axis=1, keepdims=True))
        x = t / jnp.maximum(nrm, 1e-12)
        x_ref[...] = x
        xs_ref[...] = x * dis

    return pl.pallas_call(
        body,
        grid=(N // BTC,),
        in_specs=[
            pl.BlockSpec((BTC, EMB), lambda i: (i, 0)),
            pl.BlockSpec((BTC, 16), lambda i: (i, 0)),
        ],
        out_specs=[
            pl.BlockSpec((BTC, EMB), lambda i: (i, 0)),
            pl.BlockSpec((BTC, EMB), lambda i: (i, 0)),
        ],
        out_shape=[
            jax.ShapeDtypeStruct((N, EMB), jnp.float32),
            jax.ShapeDtypeStruct((N, EMB), jnp.float32),
        ],
    )(agg, deg)


def _tc_final(E, x1, x2, x3):
    def body(e_ref, a_ref, b_ref, c_ref, o_ref):
        o_ref[...] = 0.25 * (e_ref[...] + a_ref[...] + b_ref[...]
                             + c_ref[...])

    spec = pl.BlockSpec((BTC, EMB), lambda i: (i, 0))
    return pl.pallas_call(
        body,
        grid=(N // BTC,),
        in_specs=[spec, spec, spec, spec],
        out_specs=spec,
        out_shape=jax.ShapeDtypeStruct((N, EMB), jnp.float32),
    )(E, x1, x2, x3)


def _split_cols(col, total):
    pad = total - NE
    e = jnp.arange(total, dtype=jnp.int32)
    dummy = DUMMY + (e & (DSPREAD - 1))
    colp = jnp.concatenate([col, jnp.full((pad,), -1, jnp.int32)])
    col0 = jnp.where((colp >= 0) & (colp < HALF), colp, dummy)
    col1 = jnp.where(colp >= HALF, colp - HALF, dummy)
    return jnp.stack([col0, col1])


def kernel(edge_index, E):
    row = edge_index[0]
    col = edge_index[1]

    colsd = _split_cols(col, EPD).reshape(NC, EPD // CD, CD)
    cols3 = _split_cols(col, EPC).reshape(NC, EPC // C, C)
    rows2 = jnp.concatenate(
        [row, jnp.zeros((EPC - NE,), jnp.int32)]).reshape(EPC // C, C)

    ones16 = jnp.ones((CD, 16), jnp.float32)
    z16 = jnp.zeros((STRIPE, 16), jnp.float32)
    z64 = jnp.zeros((STRIPE, EMB), jnp.float32)

    degp = _sc_deg(colsd, ones16, z16)
    deg = jnp.concatenate([degp[0, :HALF], degp[1, :HALF]], axis=0)

    xs = _tc_prescale(E, deg)
    xlist = []
    for _ in range(3):
        aggp = _sc_prop(xs, rows2, cols3, z64)
        agg = jnp.concatenate([aggp[0, :HALF], aggp[1, :HALF]], axis=0)
        x, xs = _tc_post(agg, deg)
        xlist.append(x)

    return _tc_final(E, *xlist)


# speedup vs baseline: 10.5583x; 1.4086x over previous
import functools

import jax
import jax.numpy as jnp
from jax import lax
from jax.experimental import pallas as pl
from jax.experimental.pallas import tpu as pltpu
from jax.experimental.pallas import tpu_sc as plsc

N = 50000
EMB = 64
NE = 800000

NC = 2
NS = 16
HALF = N // 2
TBL = 25088
STRIPE = TBL // NS
DUMMY = HALF
DSPREAD = 64
NBUF = 4
C = 96
M = 524
EPC = NS * C * M
CD = 128
MD = 392
NBUFD = 4
EPD = NS * CD * MD
BTC = 2000


def _mesh():
    return plsc.VectorSubcoreMesh(
        core_axis_name="c", subcore_axis_name="s", num_cores=NC,
        num_subcores=NS)


_SC_PARAMS = pltpu.CompilerParams(use_tc_tiling_on_sc=False,
                                  internal_scratch_in_bytes=131072)


def _sc_deg(colsd, ones16, z16):
    @functools.partial(
        pl.kernel,
        out_type=jax.ShapeDtypeStruct((NC, TBL, 16), jnp.float32),
        mesh=_mesh(),
        scratch_types=[
            pltpu.VMEM((NBUFD, CD), jnp.int32),
            pltpu.VMEM((CD, 16), jnp.float32),
            pltpu.VMEM_SHARED((TBL, 16), jnp.float32),
            pltpu.SemaphoreType.DMA,
            pltpu.SemaphoreType.DMA,
            pltpu.SemaphoreType.DMA,
            pltpu.SemaphoreType.DMA,
        ],
        compiler_params=_SC_PARAMS,
    )
    def k(col_hbm, ones_hbm, z_hbm, out_hbm, cidx, ones_v, table, s0,
          s1, s2, s3):
        cid = lax.axis_index("c")
        sid = lax.axis_index("s")
        ssem = [s0, s1, s2, s3]
        pltpu.sync_copy(ones_hbm, ones_v)
        pltpu.sync_copy(z_hbm, table.at[pl.ds(sid * STRIPE, STRIPE)])
        plsc.subcore_barrier()

        base = sid * MD
        for b in range(NBUFD):
            pltpu.sync_copy(col_hbm.at[cid, base + b], cidx.at[b])
            pltpu.async_copy(ones_v, table.at[cidx.at[b]], ssem[b], add=True)

        @pl.loop(1, MD // NBUFD)
        def _(it):
            for b in range(NBUFD):
                pltpu.make_async_copy(z_hbm.at[pl.ds(0, CD)], ones_v,
                                      ssem[b]).wait()
                pltpu.sync_copy(col_hbm.at[cid, base + it * NBUFD + b],
                                cidx.at[b])
                pltpu.async_copy(ones_v, table.at[cidx.at[b]], ssem[b],
                                 add=True)

        for b in range(NBUFD):
            pltpu.make_async_copy(z_hbm.at[pl.ds(0, CD)], ones_v,
                                  ssem[b]).wait()

        plsc.subcore_barrier()
        pltpu.sync_copy(table.at[pl.ds(sid * STRIPE, STRIPE)],
                        out_hbm.at[cid, pl.ds(sid * STRIPE, STRIPE)])

    return k(colsd, ones16, z16)


def _sc_prop(xs, rc, z64):
    @functools.partial(
        pl.kernel,
        out_type=jax.ShapeDtypeStruct((NC, TBL, EMB), jnp.float32),
        mesh=_mesh(),
        scratch_types=[
            pltpu.VMEM((NBUF, 2, C), jnp.int32),
            pltpu.VMEM((NBUF, C, EMB), jnp.float32),
            pltpu.VMEM_SHARED((TBL, EMB), jnp.float32),
            pltpu.SemaphoreType.DMA,
            pltpu.SemaphoreType.DMA,
            pltpu.SemaphoreType.DMA,
            pltpu.SemaphoreType.DMA,
            pltpu.SemaphoreType.DMA,
            pltpu.SemaphoreType.DMA,
            pltpu.SemaphoreType.DMA,
            pltpu.SemaphoreType.DMA,
        ],
        compiler_params=_SC_PARAMS,
    )
    def k(xs_hbm, rc_hbm, z_hbm, out_hbm, rcidx, rows_v,
          table, g0, g1, g2, g3, s0, s1, s2, s3):
        cid = lax.axis_index("c")
        sid = lax.axis_index("s")
        gsem = [g0, g1, g2, g3]
        ssem = [s0, s1, s2, s3]
        pltpu.sync_copy(z_hbm, table.at[pl.ds(sid * STRIPE, STRIPE)])
        plsc.subcore_barrier()

        base = sid * M
        for b in range(NBUF):
            pltpu.sync_copy(rc_hbm.at[cid, base + b], rcidx.at[b])
            pltpu.async_copy(xs_hbm.at[rcidx.at[b, 0]], rows_v.at[b],
                             gsem[b])

        @pl.loop(0, M // NBUF)
        def _(it):
            for b in range(NBUF):
                ci = it * NBUF + b
                prev = (b - 1) % NBUF
                pltpu.make_async_copy(xs_hbm.at[pl.ds(0, C)], rows_v.at[b],
                                      gsem[b]).wait()
                pltpu.async_copy(rows_v.at[b], table.at[rcidx.at[b, 1]],
                                 ssem[b], add=True)

                @pl.when(jnp.logical_and(ci >= 1, ci <= M - NBUF))
                def _():
                    pltpu.make_async_copy(xs_hbm.at[pl.ds(0, C)],
                                          rows_v.at[prev],
                                          ssem[prev]).wait()
                    pltpu.sync_copy(rc_hbm.at[cid, base + ci - 1 + NBUF],
                                    rcidx.at[prev])
                    pltpu.async_copy(xs_hbm.at[rcidx.at[prev, 0]],
                                     rows_v.at[prev], gsem[prev])

        for b in range(NBUF):
            pltpu.make_async_copy(xs_hbm.at[pl.ds(0, C)], rows_v.at[b],
                                  ssem[b]).wait()

        plsc.subcore_barrier()
        pltpu.sync_copy(table.at[pl.ds(sid * STRIPE, STRIPE)],
                        out_hbm.at[cid, pl.ds(sid * STRIPE, STRIPE)])

    return k(xs, rc, z64)


def _dis_block(deg_blk):
    d = deg_blk[:, 0:1]
    return jnp.where(d > 0, lax.rsqrt(d), 0.0)


def _tc_prescale(E, deg):
    def body(deg_ref, e_ref, o_ref):
        o_ref[...] = e_ref[...] * _dis_block(deg_ref[...])

    return pl.pallas_call(
        body,
        grid=(N // BTC,),
        in_specs=[
            pl.BlockSpec((BTC, 16), lambda i: (i, 0)),
            pl.BlockSpec((BTC, EMB), lambda i: (i, 0)),
        ],
        out_specs=pl.BlockSpec((BTC, EMB), lambda i: (i, 0)),
        out_shape=jax.ShapeDtypeStruct((N, EMB), jnp.float32),
    )(deg, E)


def _tc_post(agg, deg):
    def body(agg_ref, deg_ref, x_ref, xs_ref):
        dis = _dis_block(deg_ref[...])
        t = agg_ref[...] * dis
        t = jnp.where(t >= 0, t, 0.01 * t)
        nrm = jnp.sqrt(jnp.sum(t * t, axis=1, keepdims=True))
        x = t / jnp.maximum(nrm, 1e-12)
        x_ref[...] = x
        xs_ref[...] = x * dis

    return pl.pallas_call(
        body,
        grid=(N // BTC,),
        in_specs=[
            pl.BlockSpec((BTC, EMB), lambda i: (i, 0)),
            pl.BlockSpec((BTC, 16), lambda i: (i, 0)),
        ],
        out_specs=[
            pl.BlockSpec((BTC, EMB), lambda i: (i, 0)),
            pl.BlockSpec((BTC, EMB), lambda i: (i, 0)),
        ],
        out_shape=[
            jax.ShapeDtypeStruct((N, EMB), jnp.float32),
            jax.ShapeDtypeStruct((N, EMB), jnp.float32),
        ],
    )(agg, deg)


def _tc_final(E, x1, x2, x3):
    def body(e_ref, a_ref, b_ref, c_ref, o_ref):
        o_ref[...] = 0.25 * (e_ref[...] + a_ref[...] + b_ref[...]
                             + c_ref[...])

    spec = pl.BlockSpec((BTC, EMB), lambda i: (i, 0))
    return pl.pallas_call(
        body,
        grid=(N // BTC,),
        in_specs=[spec, spec, spec, spec],
        out_specs=spec,
        out_shape=jax.ShapeDtypeStruct((N, EMB), jnp.float32),
    )(E, x1, x2, x3)


def _split_cols(col, total):
    pad = total - NE
    e = jnp.arange(total, dtype=jnp.int32)
    dummy = DUMMY + (e & (DSPREAD - 1))
    colp = jnp.concatenate([col, jnp.full((pad,), -1, jnp.int32)])
    col0 = jnp.where((colp >= 0) & (colp < HALF), colp, dummy)
    col1 = jnp.where(colp >= HALF, colp - HALF, dummy)
    return jnp.stack([col0, col1])


def kernel(edge_index, E):
    row = edge_index[0]
    col = edge_index[1]

    colsd = _split_cols(col, EPD).reshape(NC, EPD // CD, CD)
    cols3 = _split_cols(col, EPC).reshape(NC, EPC // C, C)
    rows2 = jnp.concatenate(
        [row, jnp.zeros((EPC - NE,), jnp.int32)]).reshape(EPC // C, C)
    rc = jnp.stack(
        [jnp.broadcast_to(rows2, (NC, EPC // C, C)), cols3], axis=2)

    ones16 = jnp.ones((CD, 16), jnp.float32)
    z16 = jnp.zeros((STRIPE, 16), jnp.float32)
    z64 = jnp.zeros((STRIPE, EMB), jnp.float32)

    degp = _sc_deg(colsd, ones16, z16)
    deg = jnp.concatenate([degp[0, :HALF], degp[1, :HALF]], axis=0)

    xs = _tc_prescale(E, deg)
    xlist = []
    for _ in range(3):
        aggp = _sc_prop(xs, rc, z64)
        agg = jnp.concatenate([aggp[0, :HALF], aggp[1, :HALF]], axis=0)
        x, xs = _tc_post(agg, deg)
        xlist.append(x)

    return _tc_final(E, *xlist)
